# baseline (speedup 1.0000x reference)
import functools

import jax
import jax.numpy as jnp
from jax import lax
from jax.experimental import pallas as pl
from jax.experimental.pallas import tpu as pltpu
from jax.experimental.pallas import tpu_sc as plsc

_TK = 2
_LANES = 16


def _act_body(x_ref, act_ref, *, nb, nc, in_ch, inv_count):
    i = pl.program_id(0)

    @pl.when(i == 0)
    def _init():
        for c in range(nc):
            act_ref[c] = 0.0

    xa = jnp.abs(x_ref[...])
    for c in range(nc):
        act_ref[c] += jnp.sum(xa[:, c * in_ch:(c + 1) * in_ch])

    @pl.when(i == nb - 1)
    def _finalize():
        for c in range(nc):
            act_ref[c] = act_ref[c] * inv_count


def _sc_router_body(act_hbm, idx_hbm, act_v, idx_v):
    wid = lax.axis_index("s") * 2 + lax.axis_index("c")

    @pl.when(wid == 0)
    def _():
        pltpu.sync_copy(act_hbm, act_v)
        keys = act_v[...]
        vals = lax.iota(jnp.int32, _LANES)
        _, sv = plsc.sort_key_val(keys, vals, descending=True)
        idx_v[...] = sv
        pltpu.sync_copy(idx_v.at[pl.ds(0, 8)], idx_hbm)


def _sc_router(act_padded):
    mesh = plsc.VectorSubcoreMesh(core_axis_name="c", subcore_axis_name="s")
    return pl.kernel(
        _sc_router_body,
        out_type=jax.ShapeDtypeStruct((8,), jnp.int32),
        mesh=mesh,
        scratch_types=[pltpu.VMEM((_LANES,), jnp.float32),
                       pltpu.VMEM((_LANES,), jnp.int32)],
        compiler_params=pltpu.CompilerParams(needs_layout_passes=False),
    )(act_padded)


def _mm_body(idx_ref, x_ref, w_ref, b_ref, o_ref):
    c = pl.program_id(0)
    sel = (c == idx_ref[0]) | (c == idx_ref[1])

    @pl.when(sel)
    def _compute():
        acc = jax.lax.dot_general(
            x_ref[...], w_ref[0],
            dimension_numbers=(((1,), (1,)), ((), ())),
            preferred_element_type=jnp.float32)
        o_ref[...] = acc + b_ref[0]

    @pl.when(jnp.logical_not(sel))
    def _zero():
        o_ref[...] = jnp.zeros_like(o_ref)


def kernel(x, W, b):
    n_tok, in_f = x.shape
    nc, out_ch, in_ch = W.shape
    out_f = nc * out_ch

    bn1 = 2048
    nb1 = n_tok // bn1
    act = pl.pallas_call(
        functools.partial(_act_body, nb=nb1, nc=nc, in_ch=in_ch,
                          inv_count=1.0 / (n_tok * in_ch)),
        grid=(nb1,),
        in_specs=[pl.BlockSpec((bn1, in_f), lambda i: (i, 0))],
        out_specs=pl.BlockSpec(memory_space=pltpu.SMEM),
        out_shape=jax.ShapeDtypeStruct((nc,), jnp.float32),
    )(x)

    act_padded = jnp.concatenate(
        [act, jnp.full((_LANES - nc,), -1.0, dtype=act.dtype)])
    idx = _sc_router(act_padded)[:_TK]

    bn2 = 8192
    nb2 = n_tok // bn2

    def x_map(c, n, idx_ref):
        sel = (c == idx_ref[0]) | (c == idx_ref[1])
        return (jnp.where(sel, n, 0), jnp.where(sel, c, idx_ref[0]))

    def w_map(c, n, idx_ref):
        sel = (c == idx_ref[0]) | (c == idx_ref[1])
        return (jnp.where(sel, c, idx_ref[0]), 0, 0)

    def b_map(c, n, idx_ref):
        sel = (c == idx_ref[0]) | (c == idx_ref[1])
        return (jnp.where(sel, c, idx_ref[0]), 0, 0)

    out = pl.pallas_call(
        _mm_body,
        grid_spec=pltpu.PrefetchScalarGridSpec(
            num_scalar_prefetch=1,
            grid=(nc, nb2),
            in_specs=[
                pl.BlockSpec((bn2, in_ch), x_map),
                pl.BlockSpec((1, out_ch, in_ch), w_map),
                pl.BlockSpec((1, 1, out_ch), b_map),
            ],
            out_specs=pl.BlockSpec((bn2, out_ch), lambda c, n, idx_ref: (n, c)),
        ),
        out_shape=jax.ShapeDtypeStruct((n_tok, out_f), jnp.float32),
    )(idx, x, W, b.reshape(nc, 1, out_ch))

    return out, act


# speedup vs baseline: 1.0158x; 1.0158x over previous
import functools

import jax
import jax.numpy as jnp
from jax import lax
from jax.experimental import pallas as pl
from jax.experimental.pallas import tpu as pltpu
from jax.experimental.pallas import tpu_sc as plsc

_TK = 2
_LANES = 16


def _act_body(x_ref, act_ref, act16_ref, *, nb, nc, in_ch, inv_count):
    i = pl.program_id(0)

    @pl.when(i == 0)
    def _init():
        for c in range(nc):
            act_ref[c] = 0.0

    xa = jnp.abs(x_ref[...])
    for c in range(nc):
        act_ref[c] += jnp.sum(xa[:, c * in_ch:(c + 1) * in_ch])

    @pl.when(i == nb - 1)
    def _finalize():
        for c in range(nc):
            act_ref[c] = act_ref[c] * inv_count
        for c in range(_LANES):
            act16_ref[c] = act_ref[c] if c < nc else -1.0


def _sc_router_body(act_hbm, idx_hbm, act_v, idx_v):
    wid = lax.axis_index("s") * 2 + lax.axis_index("c")

    @pl.when(wid == 0)
    def _():
        pltpu.sync_copy(act_hbm, act_v)
        keys = act_v[...]
        vals = lax.iota(jnp.int32, _LANES)
        _, sv = plsc.sort_key_val(keys, vals, descending=True)
        idx_v[...] = sv
        pltpu.sync_copy(idx_v.at[pl.ds(0, 8)], idx_hbm)


def _sc_router(act_padded):
    mesh = plsc.VectorSubcoreMesh(core_axis_name="c", subcore_axis_name="s")
    return pl.kernel(
        _sc_router_body,
        out_type=jax.ShapeDtypeStruct((8,), jnp.int32),
        mesh=mesh,
        scratch_types=[pltpu.VMEM((_LANES,), jnp.float32),
                       pltpu.VMEM((_LANES,), jnp.int32)],
        compiler_params=pltpu.CompilerParams(needs_layout_passes=False),
    )(act_padded)


def _mm_body(idx_ref, x_ref, w_ref, b_ref, o_ref):
    c = pl.program_id(0)
    sel = (c == idx_ref[0]) | (c == idx_ref[1])

    @pl.when(sel)
    def _compute():
        acc = jax.lax.dot_general(
            x_ref[...], w_ref[0],
            dimension_numbers=(((1,), (1,)), ((), ())),
            preferred_element_type=jnp.float32)
        o_ref[...] = acc + b_ref[0]

    @pl.when(jnp.logical_not(sel))
    def _zero():
        o_ref[...] = jnp.zeros_like(o_ref)


def kernel(x, W, b):
    n_tok, in_f = x.shape
    nc, out_ch, in_ch = W.shape
    out_f = nc * out_ch

    bn1 = 2048
    nb1 = n_tok // bn1
    act, act16 = pl.pallas_call(
        functools.partial(_act_body, nb=nb1, nc=nc, in_ch=in_ch,
                          inv_count=1.0 / (n_tok * in_ch)),
        grid=(nb1,),
        in_specs=[pl.BlockSpec((bn1, in_f), lambda i: (i, 0))],
        out_specs=[pl.BlockSpec(memory_space=pltpu.SMEM),
                   pl.BlockSpec(memory_space=pltpu.SMEM)],
        out_shape=[jax.ShapeDtypeStruct((nc,), jnp.float32),
                   jax.ShapeDtypeStruct((_LANES,), jnp.float32)],
    )(x)

    idx = _sc_router(act16)

    bn2 = 8192
    nb2 = n_tok // bn2

    def x_map(c, n, idx_ref):
        sel = (c == idx_ref[0]) | (c == idx_ref[1])
        return (jnp.where(sel, n, 0), jnp.where(sel, c, idx_ref[0]))

    def w_map(c, n, idx_ref):
        sel = (c == idx_ref[0]) | (c == idx_ref[1])
        return (jnp.where(sel, c, idx_ref[0]), 0, 0)

    def b_map(c, n, idx_ref):
        sel = (c == idx_ref[0]) | (c == idx_ref[1])
        return (jnp.where(sel, c, idx_ref[0]), 0, 0)

    out = pl.pallas_call(
        _mm_body,
        grid_spec=pltpu.PrefetchScalarGridSpec(
            num_scalar_prefetch=1,
            grid=(nc, nb2),
            in_specs=[
                pl.BlockSpec((bn2, in_ch), x_map),
                pl.BlockSpec((1, out_ch, in_ch), w_map),
                pl.BlockSpec((1, 1, out_ch), b_map),
            ],
            out_specs=pl.BlockSpec((bn2, out_ch), lambda c, n, idx_ref: (n, c)),
        ),
        out_shape=jax.ShapeDtypeStruct((n_tok, out_f), jnp.float32),
    )(idx, x, W, b.reshape(nc, 1, out_ch))

    return out, act
